# Initial kernel scaffold; baseline (speedup 1.0000x reference)
import functools

import jax
import jax.numpy as jnp
from jax import lax
from jax.experimental import pallas as pl
from jax.experimental.pallas import tpu as pltpu
from jax.experimental.pallas import tpu_sc as plsc

NC = 2
NS = 16
NW = NC * NS
K = 80
DW = 16


def _mesh():
    return plsc.VectorSubcoreMesh(core_axis_name="c", subcore_axis_name="s")


@functools.lru_cache(maxsize=None)
def _deg_kernel(n, e):
    assert e % (NW * K) == 0 and n % NS == 0
    ew = e // NW
    nt = n // NS
    n_chunks = ew // K
    zrows = 125
    assert nt % zrows == 0

    @functools.partial(
        pl.kernel,
        out_type=jax.ShapeDtypeStruct((NC, n, DW), jnp.float32),
        mesh=_mesh(),
        scratch_types=[
            pltpu.VMEM((K,), jnp.int32),
            pltpu.VMEM((K, DW), jnp.float32),
            pltpu.VMEM((zrows, DW), jnp.float32),
            pltpu.VMEM_SHARED((n, DW), jnp.float32),
        ],
    )
    def deg(dst_hbm, out_hbm, dst_v, ones_v, zbuf, acc_sh):
        c = lax.axis_index("c")
        s = lax.axis_index("s")
        wid = c * NS + s

        def fill(i, _):
            ones_v[i, pl.ds(0, DW)] = jnp.full((DW,), 1.0, jnp.float32)
            return 0

        lax.fori_loop(0, K, fill, 0)

        def zfill(i, _):
            zbuf[i, pl.ds(0, DW)] = jnp.zeros((DW,), jnp.float32)
            return 0

        lax.fori_loop(0, zrows, zfill, 0)

        def zcopy(j, _):
            pltpu.sync_copy(zbuf, acc_sh.at[pl.ds(s * nt + j * zrows, zrows), :])
            return 0

        lax.fori_loop(0, nt // zrows, zcopy, 0)
        plsc.subcore_barrier()

        def body(i, _):
            off = wid * ew + i * K
            pltpu.sync_copy(dst_hbm.at[pl.ds(off, K)], dst_v)
            pltpu.sync_copy(ones_v, acc_sh.at[dst_v], add=True)
            return 0

        lax.fori_loop(0, n_chunks, body, 0)
        plsc.subcore_barrier()
        pltpu.sync_copy(acc_sh.at[pl.ds(s * nt, nt), :],
                        out_hbm.at[c, pl.ds(s * nt, nt), :])

    return deg


@functools.lru_cache(maxsize=None)
def _agg_kernel(n, e, d):
    assert e % (NW * K) == 0 and n % NS == 0
    ew = e // NW
    nt = n // NS
    n_chunks = ew // K
    zrows = 125
    assert nt % zrows == 0

    @functools.partial(
        pl.kernel,
        out_type=jax.ShapeDtypeStruct((NC, n, d), jnp.float32),
        mesh=_mesh(),
        scratch_types=[
            pltpu.VMEM((K,), jnp.int32),
            pltpu.VMEM((K,), jnp.int32),
            pltpu.VMEM((K, d), jnp.float32),
            pltpu.VMEM((125, d), jnp.float32),
            pltpu.VMEM_SHARED((n, d), jnp.float32),
            pltpu.SemaphoreType.DMA,
        ],
    )
    def agg(hs_hbm, src_hbm, dst_hbm, out_hbm,
            src_v, dst_v, rows_v, zbuf, acc_sh, sem):
        c = lax.axis_index("c")
        s = lax.axis_index("s")
        wid = c * NS + s
        lanes_per_row = d // 16

        def zfill(i, _):
            r = i // lanes_per_row
            col = (i % lanes_per_row) * 16
            zbuf[r, pl.ds(col, 16)] = jnp.zeros((16,), jnp.float32)
            return 0

        lax.fori_loop(0, zrows * lanes_per_row, zfill, 0)

        def zcopy(j, _):
            pltpu.sync_copy(zbuf, acc_sh.at[pl.ds(s * nt + j * zrows, zrows), :])
            return 0

        lax.fori_loop(0, nt // zrows, zcopy, 0)
        plsc.subcore_barrier()

        def body(i, _):
            off = wid * ew + i * K
            pltpu.sync_copy(src_hbm.at[pl.ds(off, K)], src_v)
            pltpu.sync_copy(dst_hbm.at[pl.ds(off, K)], dst_v)
            pltpu.async_copy(hs_hbm.at[src_v], rows_v, sem).wait()
            pltpu.sync_copy(rows_v, acc_sh.at[dst_v], add=True)
            return 0

        lax.fori_loop(0, n_chunks, body, 0)
        plsc.subcore_barrier()
        pltpu.sync_copy(acc_sh.at[pl.ds(s * nt, nt), :],
                        out_hbm.at[c, pl.ds(s * nt, nt), :])

    return agg


def _t1_body(degp_ref, x_ref, w_ref, dis_ref, hs_ref):
    deg = degp_ref[0, :, 0:1] + degp_ref[1, :, 0:1] + 1.0
    dis = lax.rsqrt(deg)
    dis_ref[...] = dis
    h = jnp.dot(x_ref[...], w_ref[...], preferred_element_type=jnp.float32)
    hs_ref[...] = h * dis


def _t2_body(aggp_ref, hs_ref, dis_ref, b_ref, w_ref, out_ref):
    dis = dis_ref[...]
    t = dis * (aggp_ref[0] + aggp_ref[1] + hs_ref[...]) + b_ref[...]
    t = jnp.maximum(t, 0.0)
    out_ref[...] = jnp.dot(t, w_ref[...], preferred_element_type=jnp.float32) * dis


def _t3_body(aggp_ref, hs_ref, dis_ref, b_ref, out_ref):
    out_ref[...] = (dis_ref[...] * (aggp_ref[0] + aggp_ref[1] + hs_ref[...])
                    + b_ref[...])


def _row_spec(b, d):
    return pl.BlockSpec((b, d), lambda i: (i, 0))


def _pair_spec(b, d):
    return pl.BlockSpec((NC, b, d), lambda i: (0, i, 0))


def _full_spec(r, c):
    return pl.BlockSpec((r, c), lambda i: (0, 0))


def _t1_call(degp, x, w, blk):
    n, d = x.shape
    return pl.pallas_call(
        _t1_body,
        grid=(n // blk,),
        in_specs=[_pair_spec(blk, DW), _row_spec(blk, d), _full_spec(d, w.shape[1])],
        out_specs=[pl.BlockSpec((blk, 1), lambda i: (i, 0)),
                   _row_spec(blk, w.shape[1])],
        out_shape=[jax.ShapeDtypeStruct((n, 1), jnp.float32),
                   jax.ShapeDtypeStruct((n, w.shape[1]), jnp.float32)],
    )(degp, x, w)


def _t2_call(aggp, hs, dis, b, w, blk):
    n, d = hs.shape
    return pl.pallas_call(
        _t2_body,
        grid=(n // blk,),
        in_specs=[_pair_spec(blk, d), _row_spec(blk, d),
                  pl.BlockSpec((blk, 1), lambda i: (i, 0)),
                  _full_spec(1, d), _full_spec(d, w.shape[1])],
        out_specs=_row_spec(blk, w.shape[1]),
        out_shape=jax.ShapeDtypeStruct((n, w.shape[1]), jnp.float32),
    )(aggp, hs, dis, b, w)


def _t3_call(aggp, hs, dis, b, blk):
    n, d = hs.shape
    return pl.pallas_call(
        _t3_body,
        grid=(n // blk,),
        in_specs=[_pair_spec(blk, d), _row_spec(blk, d),
                  pl.BlockSpec((blk, 1), lambda i: (i, 0)),
                  _full_spec(1, d)],
        out_specs=_row_spec(blk, d),
        out_shape=jax.ShapeDtypeStruct((n, d), jnp.float32),
    )(aggp, hs, dis, b)


@jax.jit
def _run(x, edge_index, W1, b1, W2, b2, W3, b3):
    src = edge_index[0]
    dst = edge_index[1]
    n = x.shape[0]
    e = edge_index.shape[1]
    blk = 1000

    degp = _deg_kernel(n, e)(dst)
    dis, hs1 = _t1_call(degp, x, W1, blk)

    agg = _agg_kernel(n, e, W1.shape[1])
    agg1 = agg(hs1, src, dst)
    hs2 = _t2_call(agg1, hs1, dis, b1.reshape(1, -1), W2, blk)
    agg2 = agg(hs2, src, dst)
    hs3 = _t2_call(agg2, hs2, dis, b2.reshape(1, -1), W3, blk)
    agg3 = agg(hs3, src, dst)
    return _t3_call(agg3, hs3, dis, b3.reshape(1, -1), blk)


def kernel(x, edge_index, W1, b1, W2, b2, W3, b3):
    return _run(x, edge_index, W1, b1, W2, b2, W3, b3)


# speedup vs baseline: 10.4499x; 10.4499x over previous
import functools

import jax
import jax.numpy as jnp
from jax import lax
from jax.experimental import pallas as pl
from jax.experimental.pallas import tpu as pltpu
from jax.experimental.pallas import tpu_sc as plsc

NC = 2
NS = 16
NW = NC * NS
K = 80
ZR = 125


def _mesh():
    return plsc.VectorSubcoreMesh(core_axis_name="c", subcore_axis_name="s")


def _zero_acc(s, nt, d, zbuf, acc_sh):
    lanes_per_row = d // 16

    def zfill(i, _):
        r = i // lanes_per_row
        col = (i % lanes_per_row) * 16
        zbuf[r, pl.ds(col, 16)] = jnp.zeros((16,), jnp.float32)
        return 0

    lax.fori_loop(0, ZR * lanes_per_row, zfill, 0)

    def zcopy(j, _):
        pltpu.sync_copy(zbuf, acc_sh.at[pl.ds(s * nt + j * ZR, ZR), :])
        return 0

    lax.fori_loop(0, nt // ZR, zcopy, 0)


@functools.lru_cache(maxsize=None)
def _deg_kernel(n, e, d):
    assert e % (NW * K) == 0 and n % NS == 0 and (n // NS) % ZR == 0
    ew = e // NW
    nt = n // NS

    @functools.partial(
        pl.kernel,
        out_type=jax.ShapeDtypeStruct((NC, NS, nt, d), jnp.float32),
        mesh=_mesh(),
        scratch_types=[
            pltpu.VMEM((1, K), jnp.int32),
            pltpu.VMEM((K, d), jnp.float32),
            pltpu.VMEM((ZR, d), jnp.float32),
            pltpu.VMEM_SHARED((n, d), jnp.float32),
        ],
    )
    def deg(dst_hbm, out_hbm, dst_v, ones_v, zbuf, acc_sh):
        c = lax.axis_index("c")
        s = lax.axis_index("s")
        wid = c * NS + s
        lanes_per_row = d // 16

        def fill(i, _):
            r = i // lanes_per_row
            col = (i % lanes_per_row) * 16
            ones_v[r, pl.ds(col, 16)] = jnp.full((16,), 1.0, jnp.float32)
            return 0

        lax.fori_loop(0, K * lanes_per_row, fill, 0)
        _zero_acc(s, nt, d, zbuf, acc_sh)
        plsc.subcore_barrier()

        def body(i, _):
            off = wid * ew + i * K
            pltpu.sync_copy(dst_hbm.at[pl.ds(off, K)], dst_v.at[0])
            pltpu.sync_copy(ones_v, acc_sh.at[dst_v.at[0]], add=True)
            return 0

        lax.fori_loop(0, ew // K, body, 0)
        plsc.subcore_barrier()
        pltpu.sync_copy(acc_sh.at[pl.ds(s * nt, nt), :], out_hbm.at[c, s])

    return deg


@functools.lru_cache(maxsize=None)
def _agg_kernel(n, e, d):
    assert e % (NW * K) == 0 and n % NS == 0 and (n // NS) % ZR == 0
    ew = e // NW
    nt = n // NS

    @functools.partial(
        pl.kernel,
        out_type=jax.ShapeDtypeStruct((NC, NS, nt, d), jnp.float32),
        mesh=_mesh(),
        scratch_types=[
            pltpu.VMEM((1, K), jnp.int32),
            pltpu.VMEM((1, K), jnp.int32),
            pltpu.VMEM((K, d), jnp.float32),
            pltpu.VMEM((ZR, d), jnp.float32),
            pltpu.VMEM_SHARED((n, d), jnp.float32),
            pltpu.SemaphoreType.DMA,
        ],
    )
    def agg(hs_hbm, src_hbm, dst_hbm, out_hbm,
            src_v, dst_v, rows_v, zbuf, acc_sh, sem):
        c = lax.axis_index("c")
        s = lax.axis_index("s")
        wid = c * NS + s
        _zero_acc(s, nt, d, zbuf, acc_sh)
        plsc.subcore_barrier()

        def body(i, _):
            off = wid * ew + i * K
            pltpu.sync_copy(src_hbm.at[pl.ds(off, K)], src_v.at[0])
            pltpu.sync_copy(dst_hbm.at[pl.ds(off, K)], dst_v.at[0])
            pltpu.async_copy(hs_hbm.at[src_v.at[0]], rows_v, sem).wait()
            pltpu.sync_copy(rows_v, acc_sh.at[dst_v.at[0]], add=True)
            return 0

        lax.fori_loop(0, ew // K, body, 0)
        plsc.subcore_barrier()
        pltpu.sync_copy(acc_sh.at[pl.ds(s * nt, nt), :], out_hbm.at[c, s])

    return agg


def _t1_body(degp_ref, x_ref, w_ref, dis_ref, hs_ref):
    deg = degp_ref[0, :, 0:1] + degp_ref[1, :, 0:1] + 1.0
    dis = lax.rsqrt(deg)
    dis_ref[...] = dis
    h = jnp.dot(x_ref[...], w_ref[...], preferred_element_type=jnp.float32)
    hs_ref[...] = h * dis


def _t2_body(aggp_ref, hs_ref, dis_ref, b_ref, w_ref, out_ref):
    dis = dis_ref[...]
    t = dis * (aggp_ref[0] + aggp_ref[1] + hs_ref[...]) + b_ref[...]
    t = jnp.maximum(t, 0.0)
    out_ref[...] = jnp.dot(t, w_ref[...], preferred_element_type=jnp.float32) * dis


def _t3_body(aggp_ref, hs_ref, dis_ref, b_ref, out_ref):
    out_ref[...] = (dis_ref[...] * (aggp_ref[0] + aggp_ref[1] + hs_ref[...])
                    + b_ref[...])


def _row_spec(b, d):
    return pl.BlockSpec((b, d), lambda i: (i, 0))


def _pair_spec(b, d):
    return pl.BlockSpec((NC, b, d), lambda i: (0, i, 0))


def _full_spec(r, c):
    return pl.BlockSpec((r, c), lambda i: (0, 0))


def _t1_call(degp, x, w, blk):
    n, d = x.shape
    return pl.pallas_call(
        _t1_body,
        grid=(n // blk,),
        in_specs=[_pair_spec(blk, degp.shape[2]), _row_spec(blk, d),
                  _full_spec(d, w.shape[1])],
        out_specs=[pl.BlockSpec((blk, 1), lambda i: (i, 0)),
                   _row_spec(blk, w.shape[1])],
        out_shape=[jax.ShapeDtypeStruct((n, 1), jnp.float32),
                   jax.ShapeDtypeStruct((n, w.shape[1]), jnp.float32)],
    )(degp, x, w)


def _t2_call(aggp, hs, dis, b, w, blk):
    n, d = hs.shape
    return pl.pallas_call(
        _t2_body,
        grid=(n // blk,),
        in_specs=[_pair_spec(blk, d), _row_spec(blk, d),
                  pl.BlockSpec((blk, 1), lambda i: (i, 0)),
                  _full_spec(1, d), _full_spec(d, w.shape[1])],
        out_specs=_row_spec(blk, w.shape[1]),
        out_shape=jax.ShapeDtypeStruct((n, w.shape[1]), jnp.float32),
    )(aggp, hs, dis, b, w)


def _t3_call(aggp, hs, dis, b, blk):
    n, d = hs.shape
    return pl.pallas_call(
        _t3_body,
        grid=(n // blk,),
        in_specs=[_pair_spec(blk, d), _row_spec(blk, d),
                  pl.BlockSpec((blk, 1), lambda i: (i, 0)),
                  _full_spec(1, d)],
        out_specs=_row_spec(blk, d),
        out_shape=jax.ShapeDtypeStruct((n, d), jnp.float32),
    )(aggp, hs, dis, b)


@jax.jit
def _run(x, edge_index, W1, b1, W2, b2, W3, b3):
    src = edge_index[0]
    dst = edge_index[1]
    n = x.shape[0]
    e = edge_index.shape[1]
    d = W1.shape[1]
    blk = 1000

    degp = _deg_kernel(n, e, d)(dst).reshape(NC, n, d)
    dis, hs1 = _t1_call(degp, x, W1, blk)

    agg = _agg_kernel(n, e, d)
    agg1 = agg(hs1, src, dst).reshape(NC, n, d)
    hs2 = _t2_call(agg1, hs1, dis, b1.reshape(1, -1), W2, blk)
    agg2 = agg(hs2, src, dst).reshape(NC, n, d)
    hs3 = _t2_call(agg2, hs2, dis, b2.reshape(1, -1), W3, blk)
    agg3 = agg(hs3, src, dst).reshape(NC, n, d)
    return _t3_call(agg3, hs3, dis, b3.reshape(1, -1), blk)


def kernel(x, edge_index, W1, b1, W2, b2, W3, b3):
    return _run(x, edge_index, W1, b1, W2, b2, W3, b3)


# speedup vs baseline: 20.7227x; 1.9831x over previous
import functools

import jax
import jax.numpy as jnp
from jax import lax
from jax.experimental import pallas as pl
from jax.experimental.pallas import tpu as pltpu
from jax.experimental.pallas import tpu_sc as plsc

NC = 2
NS = 16
NW = NC * NS
K = 80
ZR = 25


def _mesh():
    return plsc.VectorSubcoreMesh(core_axis_name="c", subcore_axis_name="s")


def _zero_acc(s, nt, d, zbuf, acc_sh):
    lanes_per_row = d // 16

    def zfill(i, _):
        r = i // lanes_per_row
        col = (i % lanes_per_row) * 16
        zbuf[r, pl.ds(col, 16)] = jnp.zeros((16,), jnp.float32)
        return 0

    lax.fori_loop(0, ZR * lanes_per_row, zfill, 0)

    def zcopy(j, _):
        pltpu.sync_copy(zbuf, acc_sh.at[pl.ds(s * nt + j * ZR, ZR), :])
        return 0

    lax.fori_loop(0, nt // ZR, zcopy, 0)


@functools.lru_cache(maxsize=None)
def _deg_kernel(n, e, d):
    assert e % (NW * K) == 0 and n % NS == 0 and (n // NS) % ZR == 0
    ew = e // NW
    nt = n // NS

    @functools.partial(
        pl.kernel,
        out_type=jax.ShapeDtypeStruct((NC, NS, nt, d), jnp.float32),
        mesh=_mesh(),
        scratch_types=[
            pltpu.VMEM((1, K), jnp.int32),
            pltpu.VMEM((K, d), jnp.float32),
            pltpu.VMEM((ZR, d), jnp.float32),
            pltpu.VMEM_SHARED((n, d), jnp.float32),
        ],
    )
    def deg(dst_hbm, out_hbm, dst_v, ones_v, zbuf, acc_sh):
        c = lax.axis_index("c")
        s = lax.axis_index("s")
        wid = c * NS + s
        lanes_per_row = d // 16

        def fill(i, _):
            r = i // lanes_per_row
            col = (i % lanes_per_row) * 16
            ones_v[r, pl.ds(col, 16)] = jnp.full((16,), 1.0, jnp.float32)
            return 0

        lax.fori_loop(0, K * lanes_per_row, fill, 0)
        _zero_acc(s, nt, d, zbuf, acc_sh)
        plsc.subcore_barrier()

        def body(i, _):
            off = wid * ew + i * K
            pltpu.sync_copy(dst_hbm.at[pl.ds(off, K)], dst_v.at[0])
            pltpu.sync_copy(ones_v, acc_sh.at[dst_v.at[0]], add=True)
            return 0

        lax.fori_loop(0, ew // K, body, 0)
        plsc.subcore_barrier()
        pltpu.sync_copy(acc_sh.at[pl.ds(s * nt, nt), :], out_hbm.at[c, s])

    return deg


NBUF = 2


@functools.lru_cache(maxsize=None)
def _agg_kernel(n, e, d):
    assert e % (NW * K) == 0 and n % NS == 0 and (n // NS) % ZR == 0
    ew = e // NW
    nt = n // NS
    ch = ew // K
    tail = ch - (ch // NBUF - 1) * NBUF
    assert tail >= NBUF

    @functools.partial(
        pl.kernel,
        out_type=jax.ShapeDtypeStruct((NC, NS, nt, d), jnp.float32),
        mesh=_mesh(),
        scratch_types=[
            pltpu.VMEM((ew,), jnp.int32),
            pltpu.VMEM((ch, K), jnp.int32),
            pltpu.VMEM((ZR, d), jnp.float32),
            pltpu.VMEM_SHARED((n, d), jnp.float32),
        ]
        + [pltpu.VMEM((K, d), jnp.float32)] * NBUF
        + [pltpu.SemaphoreType.DMA] * NBUF,
    )
    def agg(hs_hbm, src_hbm, dst_hbm, out_hbm,
            src_v, dst_v, zbuf, acc_sh, *rest):
        rows = rest[:NBUF]
        sems = rest[NBUF:]
        c = lax.axis_index("c")
        s = lax.axis_index("s")
        wid = c * NS + s
        pltpu.sync_copy(src_hbm.at[wid], src_v)
        pltpu.sync_copy(dst_hbm.at[wid], dst_v)
        _zero_acc(s, nt, d, zbuf, acc_sh)
        plsc.subcore_barrier()

        for b in range(NBUF):
            pltpu.async_copy(
                hs_hbm.at[src_v.at[pl.ds(b * K, K)]], rows[b], sems[b])

        def body(g, _):
            for b in range(NBUF):
                i = g * NBUF + b
                pltpu.make_async_copy(
                    hs_hbm.at[pl.ds(0, K)], rows[b], sems[b]).wait()
                pltpu.sync_copy(rows[b], acc_sh.at[dst_v.at[i]], add=True)
                pltpu.async_copy(
                    hs_hbm.at[src_v.at[pl.ds((i + NBUF) * K, K)]],
                    rows[b], sems[b])
            return 0

        lax.fori_loop(0, ch // NBUF - 1, body, 0)
        base = (ch // NBUF - 1) * NBUF
        for t in range(tail):
            b = t % NBUF
            i = base + t
            pltpu.make_async_copy(
                hs_hbm.at[pl.ds(0, K)], rows[b], sems[b]).wait()
            pltpu.sync_copy(rows[b], acc_sh.at[dst_v.at[i]], add=True)
            if i + NBUF < ch:
                pltpu.async_copy(
                    hs_hbm.at[src_v.at[pl.ds((i + NBUF) * K, K)]],
                    rows[b], sems[b])

        plsc.subcore_barrier()
        pltpu.sync_copy(acc_sh.at[pl.ds(s * nt, nt), :], out_hbm.at[c, s])

    return agg


def _t1_body(degp_ref, x_ref, w_ref, dis_ref, hs_ref):
    deg = degp_ref[0, :, 0:1] + degp_ref[1, :, 0:1] + 1.0
    dis = lax.rsqrt(deg)
    dis_ref[...] = dis
    h = jnp.dot(x_ref[...], w_ref[...], preferred_element_type=jnp.float32)
    hs_ref[...] = h * dis


def _t2_body(aggp_ref, hs_ref, dis_ref, b_ref, w_ref, out_ref):
    dis = dis_ref[...]
    t = dis * (aggp_ref[0] + aggp_ref[1] + hs_ref[...]) + b_ref[...]
    t = jnp.maximum(t, 0.0)
    out_ref[...] = jnp.dot(t, w_ref[...], preferred_element_type=jnp.float32) * dis


def _t3_body(aggp_ref, hs_ref, dis_ref, b_ref, out_ref):
    out_ref[...] = (dis_ref[...] * (aggp_ref[0] + aggp_ref[1] + hs_ref[...])
                    + b_ref[...])


def _row_spec(b, d):
    return pl.BlockSpec((b, d), lambda i: (i, 0))


def _pair_spec(b, d):
    return pl.BlockSpec((NC, b, d), lambda i: (0, i, 0))


def _full_spec(r, c):
    return pl.BlockSpec((r, c), lambda i: (0, 0))


def _t1_call(degp, x, w, blk):
    n, d = x.shape
    return pl.pallas_call(
        _t1_body,
        grid=(n // blk,),
        in_specs=[_pair_spec(blk, degp.shape[2]), _row_spec(blk, d),
                  _full_spec(d, w.shape[1])],
        out_specs=[pl.BlockSpec((blk, 1), lambda i: (i, 0)),
                   _row_spec(blk, w.shape[1])],
        out_shape=[jax.ShapeDtypeStruct((n, 1), jnp.float32),
                   jax.ShapeDtypeStruct((n, w.shape[1]), jnp.float32)],
    )(degp, x, w)


def _t2_call(aggp, hs, dis, b, w, blk):
    n, d = hs.shape
    return pl.pallas_call(
        _t2_body,
        grid=(n // blk,),
        in_specs=[_pair_spec(blk, d), _row_spec(blk, d),
                  pl.BlockSpec((blk, 1), lambda i: (i, 0)),
                  _full_spec(1, d), _full_spec(d, w.shape[1])],
        out_specs=_row_spec(blk, w.shape[1]),
        out_shape=jax.ShapeDtypeStruct((n, w.shape[1]), jnp.float32),
    )(aggp, hs, dis, b, w)


def _t3_call(aggp, hs, dis, b, blk):
    n, d = hs.shape
    return pl.pallas_call(
        _t3_body,
        grid=(n // blk,),
        in_specs=[_pair_spec(blk, d), _row_spec(blk, d),
                  pl.BlockSpec((blk, 1), lambda i: (i, 0)),
                  _full_spec(1, d)],
        out_specs=_row_spec(blk, d),
        out_shape=jax.ShapeDtypeStruct((n, d), jnp.float32),
    )(aggp, hs, dis, b)


@jax.jit
def _run(x, edge_index, W1, b1, W2, b2, W3, b3):
    src = edge_index[0]
    dst = edge_index[1]
    n = x.shape[0]
    e = edge_index.shape[1]
    d = W1.shape[1]
    blk = 1000

    ch = e // (NW * K)
    src3 = src.reshape(NW, e // NW)
    dst3 = dst.reshape(NW, ch, K)

    degp = _deg_kernel(n, e, d)(dst).reshape(NC, n, d)
    dis, hs1 = _t1_call(degp, x, W1, blk)

    agg = _agg_kernel(n, e, d)
    agg1 = agg(hs1, src3, dst3).reshape(NC, n, d)
    hs2 = _t2_call(agg1, hs1, dis, b1.reshape(1, -1), W2, blk)
    agg2 = agg(hs2, src3, dst3).reshape(NC, n, d)
    hs3 = _t2_call(agg2, hs2, dis, b2.reshape(1, -1), W3, blk)
    agg3 = agg(hs3, src3, dst3).reshape(NC, n, d)
    return _t3_call(agg3, hs3, dis, b3.reshape(1, -1), blk)


def kernel(x, edge_index, W1, b1, W2, b2, W3, b3):
    return _run(x, edge_index, W1, b1, W2, b2, W3, b3)


# speedup vs baseline: 25.5670x; 1.2338x over previous
import functools

import jax
import jax.numpy as jnp
from jax import lax
from jax.experimental import pallas as pl
from jax.experimental.pallas import tpu as pltpu
from jax.experimental.pallas import tpu_sc as plsc

NC = 2
NS = 16
NW = NC * NS
K = 80
ZR = 25


def _mesh():
    return plsc.VectorSubcoreMesh(core_axis_name="c", subcore_axis_name="s")


def _zero_acc(s, nt, d, zbuf, acc_sh):
    lanes_per_row = d // 16

    def zfill(i, _):
        r = i // lanes_per_row
        col = (i % lanes_per_row) * 16
        zbuf[r, pl.ds(col, 16)] = jnp.zeros((16,), jnp.float32)
        return 0

    lax.fori_loop(0, ZR * lanes_per_row, zfill, 0)

    def zcopy(j, _):
        pltpu.sync_copy(zbuf, acc_sh.at[pl.ds(s * nt + j * ZR, ZR), :])
        return 0

    lax.fori_loop(0, nt // ZR, zcopy, 0)


HR = 80


@functools.lru_cache(maxsize=None)
def _deg_kernel(n, e):
    assert e % (NW * 16) == 0 and n <= HR * 128 and HR % NS == 0
    ew = e // NW
    hrt = HR // NS

    @functools.partial(
        pl.kernel,
        out_type=jax.ShapeDtypeStruct((NC, NS, HR, 128), jnp.float32),
        mesh=_mesh(),
        compiler_params=pltpu.CompilerParams(needs_layout_passes=False),
        scratch_types=[
            pltpu.VMEM((ew,), jnp.int32),
            pltpu.VMEM((HR, 128), jnp.float32),
            pltpu.VMEM((1, HR), jnp.int32),
            pltpu.VMEM_SHARED((HR, 128), jnp.float32),
        ],
    )
    def deg(dst_hbm, out_hbm, dst_v, hist, rid, acc_sh):
        c = lax.axis_index("c")
        s = lax.axis_index("s")
        wid = c * NS + s
        pltpu.sync_copy(dst_hbm.at[wid], dst_v)

        def zfill(i, _):
            hist[i // 8, pl.ds((i % 8) * 16, 16)] = jnp.zeros((16,), jnp.float32)
            return 0

        lax.fori_loop(0, HR * 8, zfill, 0)
        base = lax.iota(jnp.int32, 16)

        def rfill(i, _):
            rid[0, pl.ds(i * 16, 16)] = base + i * 16
            return 0

        lax.fori_loop(0, HR // 16, rfill, 0)
        pltpu.sync_copy(hist.at[pl.ds(0, hrt)],
                        acc_sh.at[pl.ds(s * hrt, hrt)])
        ones = jnp.full((16,), 1.0, jnp.float32)
        plsc.subcore_barrier()

        def body(i, _):
            idx = dst_v[pl.ds(i * 16, 16)]
            plsc.addupdate_scatter(
                hist, [lax.shift_right_logical(idx, 7),
                       lax.bitwise_and(idx, 127)], ones)
            return 0

        lax.fori_loop(0, ew // 16, body, 0)
        pltpu.sync_copy(hist, acc_sh.at[rid.at[0]], add=True)
        plsc.subcore_barrier()
        pltpu.sync_copy(acc_sh, out_hbm.at[c, s])

    return deg


NBUF = 2


@functools.lru_cache(maxsize=None)
def _agg_kernel(n, e, d):
    assert e % (NW * K) == 0 and n % NS == 0 and (n // NS) % ZR == 0
    ew = e // NW
    nt = n // NS
    ch = ew // K
    tail = ch - (ch // NBUF - 1) * NBUF
    assert tail >= NBUF

    @functools.partial(
        pl.kernel,
        out_type=jax.ShapeDtypeStruct((NC, NS, nt, d), jnp.float32),
        mesh=_mesh(),
        scratch_types=[
            pltpu.VMEM((ew,), jnp.int32),
            pltpu.VMEM((ch, K), jnp.int32),
            pltpu.VMEM((ZR, d), jnp.float32),
            pltpu.VMEM_SHARED((n, d), jnp.float32),
        ]
        + [pltpu.VMEM((K, d), jnp.float32)] * NBUF
        + [pltpu.SemaphoreType.DMA] * NBUF,
    )
    def agg(hs_hbm, src_hbm, dst_hbm, out_hbm,
            src_v, dst_v, zbuf, acc_sh, *rest):
        rows = rest[:NBUF]
        sems = rest[NBUF:]
        c = lax.axis_index("c")
        s = lax.axis_index("s")
        wid = c * NS + s
        pltpu.sync_copy(src_hbm.at[wid], src_v)
        pltpu.sync_copy(dst_hbm.at[wid], dst_v)
        _zero_acc(s, nt, d, zbuf, acc_sh)
        plsc.subcore_barrier()

        for b in range(NBUF):
            pltpu.async_copy(
                hs_hbm.at[src_v.at[pl.ds(b * K, K)]], rows[b], sems[b])

        def body(g, _):
            for b in range(NBUF):
                i = g * NBUF + b
                pltpu.make_async_copy(
                    hs_hbm.at[pl.ds(0, K)], rows[b], sems[b]).wait()
                pltpu.sync_copy(rows[b], acc_sh.at[dst_v.at[i]], add=True)
                pltpu.async_copy(
                    hs_hbm.at[src_v.at[pl.ds((i + NBUF) * K, K)]],
                    rows[b], sems[b])
            return 0

        lax.fori_loop(0, ch // NBUF - 1, body, 0)
        base = (ch // NBUF - 1) * NBUF
        for t in range(tail):
            b = t % NBUF
            i = base + t
            pltpu.make_async_copy(
                hs_hbm.at[pl.ds(0, K)], rows[b], sems[b]).wait()
            pltpu.sync_copy(rows[b], acc_sh.at[dst_v.at[i]], add=True)
            if i + NBUF < ch:
                pltpu.async_copy(
                    hs_hbm.at[src_v.at[pl.ds((i + NBUF) * K, K)]],
                    rows[b], sems[b])

        plsc.subcore_barrier()
        pltpu.sync_copy(acc_sh.at[pl.ds(s * nt, nt), :], out_hbm.at[c, s])

    return agg


def _t1_body(deg0_ref, deg1_ref, x_ref, w_ref, dis_ref, hs_ref):
    deg = deg0_ref[...] + deg1_ref[...] + 1.0
    dis = lax.rsqrt(deg)
    dis_ref[...] = dis
    h = jnp.dot(x_ref[...], w_ref[...], preferred_element_type=jnp.float32)
    hs_ref[...] = h * dis


def _t2_body(aggp_ref, hs_ref, dis_ref, b_ref, w_ref, out_ref):
    dis = dis_ref[...]
    t = dis * (aggp_ref[0] + aggp_ref[1] + hs_ref[...]) + b_ref[...]
    t = jnp.maximum(t, 0.0)
    out_ref[...] = jnp.dot(t, w_ref[...], preferred_element_type=jnp.float32) * dis


def _t3_body(aggp_ref, hs_ref, dis_ref, b_ref, out_ref):
    out_ref[...] = (dis_ref[...] * (aggp_ref[0] + aggp_ref[1] + hs_ref[...])
                    + b_ref[...])


def _row_spec(b, d):
    return pl.BlockSpec((b, d), lambda i: (i, 0))


def _pair_spec(b, d):
    return pl.BlockSpec((NC, b, d), lambda i: (0, i, 0))


def _full_spec(r, c):
    return pl.BlockSpec((r, c), lambda i: (0, 0))


def _t1_call(deg0, deg1, x, w, blk):
    n, d = x.shape
    col = pl.BlockSpec((blk, 1), lambda i: (i, 0))
    return pl.pallas_call(
        _t1_body,
        grid=(n // blk,),
        in_specs=[col, col, _row_spec(blk, d), _full_spec(d, w.shape[1])],
        out_specs=[col, _row_spec(blk, w.shape[1])],
        out_shape=[jax.ShapeDtypeStruct((n, 1), jnp.float32),
                   jax.ShapeDtypeStruct((n, w.shape[1]), jnp.float32)],
    )(deg0, deg1, x, w)


def _t2_call(aggp, hs, dis, b, w, blk):
    n, d = hs.shape
    return pl.pallas_call(
        _t2_body,
        grid=(n // blk,),
        in_specs=[_pair_spec(blk, d), _row_spec(blk, d),
                  pl.BlockSpec((blk, 1), lambda i: (i, 0)),
                  _full_spec(1, d), _full_spec(d, w.shape[1])],
        out_specs=_row_spec(blk, w.shape[1]),
        out_shape=jax.ShapeDtypeStruct((n, w.shape[1]), jnp.float32),
    )(aggp, hs, dis, b, w)


def _t3_call(aggp, hs, dis, b, blk):
    n, d = hs.shape
    return pl.pallas_call(
        _t3_body,
        grid=(n // blk,),
        in_specs=[_pair_spec(blk, d), _row_spec(blk, d),
                  pl.BlockSpec((blk, 1), lambda i: (i, 0)),
                  _full_spec(1, d)],
        out_specs=_row_spec(blk, d),
        out_shape=jax.ShapeDtypeStruct((n, d), jnp.float32),
    )(aggp, hs, dis, b)


@jax.jit
def _run(x, edge_index, W1, b1, W2, b2, W3, b3):
    src = edge_index[0]
    dst = edge_index[1]
    n = x.shape[0]
    e = edge_index.shape[1]
    d = W1.shape[1]
    blk = 1000

    ch = e // (NW * K)
    src3 = src.reshape(NW, e // NW)
    dst3 = dst.reshape(NW, ch, K)

    degp = _deg_kernel(n, e)(dst.reshape(NW, e // NW))
    deg0 = degp[0, 0].reshape(HR * 128, 1)[:n]
    deg1 = degp[1, 0].reshape(HR * 128, 1)[:n]
    dis, hs1 = _t1_call(deg0, deg1, x, W1, blk)

    agg = _agg_kernel(n, e, d)
    agg1 = agg(hs1, src3, dst3).reshape(NC, n, d)
    hs2 = _t2_call(agg1, hs1, dis, b1.reshape(1, -1), W2, blk)
    agg2 = agg(hs2, src3, dst3).reshape(NC, n, d)
    hs3 = _t2_call(agg2, hs2, dis, b2.reshape(1, -1), W3, blk)
    agg3 = agg(hs3, src3, dst3).reshape(NC, n, d)
    return _t3_call(agg3, hs3, dis, b3.reshape(1, -1), blk)


def kernel(x, edge_index, W1, b1, W2, b2, W3, b3):
    return _run(x, edge_index, W1, b1, W2, b2, W3, b3)


# speedup vs baseline: 25.7380x; 1.0067x over previous
import functools

import jax
import jax.numpy as jnp
from jax import lax
from jax.experimental import pallas as pl
from jax.experimental.pallas import tpu as pltpu
from jax.experimental.pallas import tpu_sc as plsc

NC = 2
NS = 16
NW = NC * NS
K = 80
ZR = 25


def _mesh():
    return plsc.VectorSubcoreMesh(core_axis_name="c", subcore_axis_name="s")


HR = 80


@functools.lru_cache(maxsize=None)
def _deg_kernel(n, e):
    assert e % (NW * 16) == 0 and n <= HR * 128 and HR % NS == 0
    ew = e // NW
    hrt = HR // NS

    @functools.partial(
        pl.kernel,
        out_type=jax.ShapeDtypeStruct((NC, NS, HR, 128), jnp.float32),
        mesh=_mesh(),
        compiler_params=pltpu.CompilerParams(needs_layout_passes=False),
        scratch_types=[
            pltpu.VMEM((ew,), jnp.int32),
            pltpu.VMEM((HR, 128), jnp.float32),
            pltpu.VMEM((1, HR), jnp.int32),
            pltpu.VMEM_SHARED((HR, 128), jnp.float32),
        ],
    )
    def deg(dst_hbm, out_hbm, dst_v, hist, rid, acc_sh):
        c = lax.axis_index("c")
        s = lax.axis_index("s")
        wid = c * NS + s
        pltpu.sync_copy(dst_hbm.at[wid], dst_v)

        def zfill(i, _):
            hist[i // 8, pl.ds((i % 8) * 16, 16)] = jnp.zeros((16,), jnp.float32)
            return 0

        lax.fori_loop(0, HR * 8, zfill, 0)
        base = lax.iota(jnp.int32, 16)

        def rfill(i, _):
            rid[0, pl.ds(i * 16, 16)] = base + i * 16
            return 0

        lax.fori_loop(0, HR // 16, rfill, 0)
        pltpu.sync_copy(hist.at[pl.ds(0, hrt)],
                        acc_sh.at[pl.ds(s * hrt, hrt)])
        ones = jnp.full((16,), 1.0, jnp.float32)
        plsc.subcore_barrier()

        def body(i, _):
            idx = dst_v[pl.ds(i * 16, 16)]
            plsc.addupdate_scatter(
                hist, [lax.shift_right_logical(idx, 7),
                       lax.bitwise_and(idx, 127)], ones)
            return 0

        lax.fori_loop(0, ew // 16, body, 0)
        pltpu.sync_copy(hist, acc_sh.at[rid.at[0]], add=True)
        plsc.subcore_barrier()
        pltpu.sync_copy(acc_sh, out_hbm.at[c, s])

    return deg


NBUF = 2


@functools.lru_cache(maxsize=None)
def _agg_kernel(n, e, d):
    assert e % (NW * K) == 0 and n % NS == 0 and (n // NS) % ZR == 0
    ew = e // NW
    nt = n // NS
    ch = ew // K
    tail = ch - (ch // NBUF - 1) * NBUF
    assert tail >= NBUF

    @functools.partial(
        pl.kernel,
        out_type=jax.ShapeDtypeStruct((NC, NS, nt, d), jnp.float32),
        mesh=_mesh(),
        scratch_types=[
            pltpu.VMEM((ew,), jnp.int32),
            pltpu.VMEM((ch, K), jnp.int32),
            pltpu.VMEM_SHARED((n, d), jnp.float32),
        ]
        + [pltpu.VMEM((K, d), jnp.float32)] * NBUF
        + [pltpu.SemaphoreType.DMA] * NBUF,
    )
    def agg(hs_hbm, src_hbm, dst_hbm, zeros_hbm, out_hbm,
            src_v, dst_v, acc_sh, *rest):
        rows = rest[:NBUF]
        sems = rest[NBUF:]
        c = lax.axis_index("c")
        s = lax.axis_index("s")
        wid = c * NS + s
        pltpu.sync_copy(src_hbm.at[wid], src_v)
        pltpu.sync_copy(dst_hbm.at[wid], dst_v)
        pltpu.sync_copy(zeros_hbm, acc_sh.at[pl.ds(s * nt, nt), :])
        plsc.subcore_barrier()

        for b in range(NBUF):
            pltpu.async_copy(
                hs_hbm.at[src_v.at[pl.ds(b * K, K)]], rows[b], sems[b])

        def body(g, _):
            for b in range(NBUF):
                i = g * NBUF + b
                pltpu.make_async_copy(
                    hs_hbm.at[pl.ds(0, K)], rows[b], sems[b]).wait()
                pltpu.sync_copy(rows[b], acc_sh.at[dst_v.at[i]], add=True)
                pltpu.async_copy(
                    hs_hbm.at[src_v.at[pl.ds((i + NBUF) * K, K)]],
                    rows[b], sems[b])
            return 0

        lax.fori_loop(0, ch // NBUF - 1, body, 0)
        base = (ch // NBUF - 1) * NBUF
        for t in range(tail):
            b = t % NBUF
            i = base + t
            pltpu.make_async_copy(
                hs_hbm.at[pl.ds(0, K)], rows[b], sems[b]).wait()
            pltpu.sync_copy(rows[b], acc_sh.at[dst_v.at[i]], add=True)
            if i + NBUF < ch:
                pltpu.async_copy(
                    hs_hbm.at[src_v.at[pl.ds((i + NBUF) * K, K)]],
                    rows[b], sems[b])

        plsc.subcore_barrier()
        pltpu.sync_copy(acc_sh.at[pl.ds(s * nt, nt), :], out_hbm.at[c, s])

    return agg


def _t1_body(deg0_ref, deg1_ref, x_ref, w_ref, dis_ref, hs_ref):
    deg = deg0_ref[...] + deg1_ref[...] + 1.0
    dis = lax.rsqrt(deg)
    dis_ref[...] = dis
    h = jnp.dot(x_ref[...], w_ref[...], preferred_element_type=jnp.float32)
    hs_ref[...] = h * dis


def _t2_body(aggp_ref, hs_ref, dis_ref, b_ref, w_ref, out_ref):
    dis = dis_ref[...]
    t = dis * (aggp_ref[0] + aggp_ref[1] + hs_ref[...]) + b_ref[...]
    t = jnp.maximum(t, 0.0)
    out_ref[...] = jnp.dot(t, w_ref[...], preferred_element_type=jnp.float32) * dis


def _t3_body(aggp_ref, hs_ref, dis_ref, b_ref, out_ref):
    out_ref[...] = (dis_ref[...] * (aggp_ref[0] + aggp_ref[1] + hs_ref[...])
                    + b_ref[...])


def _row_spec(b, d):
    return pl.BlockSpec((b, d), lambda i: (i, 0))


def _pair_spec(b, d):
    return pl.BlockSpec((NC, b, d), lambda i: (0, i, 0))


def _full_spec(r, c):
    return pl.BlockSpec((r, c), lambda i: (0, 0))


def _t1_call(deg0, deg1, x, w, blk):
    n, d = x.shape
    col = pl.BlockSpec((blk, 1), lambda i: (i, 0))
    return pl.pallas_call(
        _t1_body,
        grid=(n // blk,),
        in_specs=[col, col, _row_spec(blk, d), _full_spec(d, w.shape[1])],
        out_specs=[col, _row_spec(blk, w.shape[1])],
        out_shape=[jax.ShapeDtypeStruct((n, 1), jnp.float32),
                   jax.ShapeDtypeStruct((n, w.shape[1]), jnp.float32)],
    )(deg0, deg1, x, w)


def _t2_call(aggp, hs, dis, b, w, blk):
    n, d = hs.shape
    return pl.pallas_call(
        _t2_body,
        grid=(n // blk,),
        in_specs=[_pair_spec(blk, d), _row_spec(blk, d),
                  pl.BlockSpec((blk, 1), lambda i: (i, 0)),
                  _full_spec(1, d), _full_spec(d, w.shape[1])],
        out_specs=_row_spec(blk, w.shape[1]),
        out_shape=jax.ShapeDtypeStruct((n, w.shape[1]), jnp.float32),
    )(aggp, hs, dis, b, w)


def _t3_call(aggp, hs, dis, b, blk):
    n, d = hs.shape
    return pl.pallas_call(
        _t3_body,
        grid=(n // blk,),
        in_specs=[_pair_spec(blk, d), _row_spec(blk, d),
                  pl.BlockSpec((blk, 1), lambda i: (i, 0)),
                  _full_spec(1, d)],
        out_specs=_row_spec(blk, d),
        out_shape=jax.ShapeDtypeStruct((n, d), jnp.float32),
    )(aggp, hs, dis, b)


@jax.jit
def _run(x, edge_index, W1, b1, W2, b2, W3, b3):
    src = edge_index[0]
    dst = edge_index[1]
    n = x.shape[0]
    e = edge_index.shape[1]
    d = W1.shape[1]
    blk = 2000

    ch = e // (NW * K)
    src3 = src.reshape(NW, e // NW)
    dst3 = dst.reshape(NW, ch, K)
    zeros = jnp.zeros((n // NS, d), jnp.float32)

    degp = _deg_kernel(n, e)(dst.reshape(NW, e // NW))
    deg0 = degp[0, 0].reshape(HR * 128, 1)
    deg1 = degp[1, 0].reshape(HR * 128, 1)
    dis, hs1 = _t1_call(deg0, deg1, x, W1, blk)

    agg = _agg_kernel(n, e, d)
    agg1 = agg(hs1, src3, dst3, zeros).reshape(NC, n, d)
    hs2 = _t2_call(agg1, hs1, dis, b1.reshape(1, -1), W2, blk)
    agg2 = agg(hs2, src3, dst3, zeros).reshape(NC, n, d)
    hs3 = _t2_call(agg2, hs2, dis, b2.reshape(1, -1), W3, blk)
    agg3 = agg(hs3, src3, dst3, zeros).reshape(NC, n, d)
    return _t3_call(agg3, hs3, dis, b3.reshape(1, -1), blk)


def kernel(x, edge_index, W1, b1, W2, b2, W3, b3):
    return _run(x, edge_index, W1, b1, W2, b2, W3, b3)
